# Initial kernel scaffold; baseline (speedup 1.0000x reference)
import functools

import jax
import jax.numpy as jnp
from jax import lax
from jax.experimental import pallas as pl
from jax.experimental.pallas import tpu as pltpu
from jax.experimental.pallas import tpu_sc as plsc

_B, _L, _D = 4096, 200, 64
_N = _B * _L
_NC, _NS = 2, 16
_NW = _NC * _NS
_PER_W = _N // _NW
_CHUNK = 128
_NCH = _PER_W // _CHUNK

_mesh = plsc.VectorSubcoreMesh(core_axis_name="c", subcore_axis_name="s")


@functools.partial(
    pl.kernel,
    out_type=jax.ShapeDtypeStruct((_N, _D), jnp.float32),
    mesh=_mesh,
    scratch_types=[
        pltpu.VMEM((_PER_W,), jnp.int32),
        pltpu.VMEM((_CHUNK, _D), jnp.float32),
        pltpu.SemaphoreType.DMA,
    ],
)
def _gather_kernel(x_hbm, table_hbm, out_hbm, idx_v, rows_v, sem):
    wid = lax.axis_index("s") * _NC + lax.axis_index("c")
    base = pl.multiple_of(wid * _PER_W, _PER_W)
    pltpu.sync_copy(x_hbm.at[pl.ds(base, _PER_W)], idx_v)

    def step(j, carry):
        off = pl.multiple_of(j * _CHUNK, _CHUNK)
        pltpu.async_copy(
            table_hbm.at[idx_v.at[pl.ds(off, _CHUNK)]], rows_v, sem
        ).wait()
        pltpu.sync_copy(rows_v, out_hbm.at[pl.ds(base + off, _CHUNK)])
        return carry

    lax.fori_loop(0, _NCH, step, 0)


def kernel(x, table):
    flat = x.reshape(_N).astype(jnp.int32)
    out = _gather_kernel(flat, table)
    return out.reshape(_B, _L, _D)


# speedup vs baseline: 3.5428x; 3.5428x over previous
import functools

import jax
import jax.numpy as jnp
from jax import lax
from jax.experimental import pallas as pl
from jax.experimental.pallas import tpu as pltpu
from jax.experimental.pallas import tpu_sc as plsc

_B, _L, _D = 4096, 200, 64
_N = _B * _L
_NC, _NS = 2, 16
_NW = _NC * _NS
_PER_W = _N // _NW
_CHUNK = 128
_NCH = _PER_W // _CHUNK

_mesh = plsc.VectorSubcoreMesh(core_axis_name="c", subcore_axis_name="s")


@functools.partial(
    pl.kernel,
    out_type=jax.ShapeDtypeStruct((_N, _D), jnp.float32),
    mesh=_mesh,
    scratch_types=[
        pltpu.VMEM((_PER_W,), jnp.int32),
        pltpu.VMEM((_CHUNK, _D), jnp.float32),
        pltpu.SemaphoreType.DMA,
    ],
    compiler_params=pltpu.CompilerParams(use_tc_tiling_on_sc=False),
)
def _gather_kernel(x_hbm, table_hbm, out_hbm, idx_v, rows_v, sem):
    wid = lax.axis_index("s") * _NC + lax.axis_index("c")
    base = pl.multiple_of(wid * _PER_W, _PER_W)
    pltpu.sync_copy(x_hbm.at[pl.ds(base, _PER_W)], idx_v)

    def step(j, carry):
        off = pl.multiple_of(j * _CHUNK, _CHUNK)
        pltpu.async_copy(
            table_hbm.at[idx_v.at[pl.ds(off, _CHUNK)]], rows_v, sem
        ).wait()
        pltpu.sync_copy(rows_v, out_hbm.at[pl.ds(base + off, _CHUNK)])
        return carry

    lax.fori_loop(0, _NCH, step, 0)


def kernel(x, table):
    flat = x.reshape(_N).astype(jnp.int32)
    out = _gather_kernel(flat, table)
    return out.reshape(_B, _L, _D)


# speedup vs baseline: 4.2613x; 1.2028x over previous
import functools

import jax
import jax.numpy as jnp
from jax import lax
from jax.experimental import pallas as pl
from jax.experimental.pallas import tpu as pltpu
from jax.experimental.pallas import tpu_sc as plsc

_B, _L, _D = 4096, 200, 64
_N = _B * _L
_NC, _NS = 2, 16
_NW = _NC * _NS
_PER_W = _N // _NW
_CHUNK = 128
_NCH = _PER_W // _CHUNK
_NBUF = 8
_ROUNDS = _NCH // _NBUF

_mesh = plsc.VectorSubcoreMesh(core_axis_name="c", subcore_axis_name="s")


@functools.partial(
    pl.kernel,
    out_type=jax.ShapeDtypeStruct((_N, _D), jnp.float32),
    mesh=_mesh,
    scratch_types=(
        [pltpu.VMEM((_NCH, _CHUNK), jnp.int32)]
        + [pltpu.VMEM((_CHUNK, _D), jnp.float32) for _ in range(_NBUF)]
        + [pltpu.SemaphoreType.DMA for _ in range(2 * _NBUF)]
    ),
    compiler_params=pltpu.CompilerParams(use_tc_tiling_on_sc=False),
)
def _gather_kernel(x_hbm, table_hbm, out_hbm, idx_v, *rest):
    bufs = rest[:_NBUF]
    gsem = rest[_NBUF:2 * _NBUF]
    wsem = rest[2 * _NBUF:]

    wid = lax.axis_index("s") * _NC + lax.axis_index("c")
    base = pl.multiple_of(wid * _PER_W, _PER_W)
    pltpu.sync_copy(x_hbm.at[wid], idx_v)

    def gather(j, b):
        return pltpu.make_async_copy(
            table_hbm.at[idx_v.at[j]], bufs[b], gsem[b])

    def write(j, b):
        off = pl.multiple_of(j * _CHUNK, _CHUNK)
        return pltpu.make_async_copy(
            bufs[b], out_hbm.at[pl.ds(base + off, _CHUNK)], wsem[b])

    for b in range(_NBUF):
        gather(b, b).start()

    def round_body(r, carry):
        j0 = r * _NBUF
        for b in range(_NBUF):
            gather(j0 + b, b).wait()
            write(j0 + b, b).start()
        for b in range(_NBUF):
            write(j0 + b, b).wait()
            gather(j0 + _NBUF + b, b).start()
        return carry

    lax.fori_loop(0, _ROUNDS - 1, round_body, 0)

    j0 = (_ROUNDS - 1) * _NBUF
    for b in range(_NBUF):
        gather(j0 + b, b).wait()
        write(j0 + b, b).start()
    for b in range(_NBUF):
        write(j0 + b, b).wait()


def kernel(x, table):
    idx = x.reshape(_NW, _NCH, _CHUNK).astype(jnp.int32)
    out = _gather_kernel(idx, table)
    return out.reshape(_B, _L, _D)
